# baseline (speedup 1.0000x reference)
import jax
import jax.numpy as jnp
from jax import lax
from jax.experimental import pallas as pl
from jax.experimental.pallas import tpu as pltpu
from jax.experimental.pallas import tpu_sc as plsc

N = 10000
D = 128
NC = 2
NS = 16
NW = NC * NS
EPB = 128
NPAD = 10112
RT = NPAD // NS
DW = 8


def _zero_chunks(total):
  out, off = [], 0
  while off < total:
    step = min(EPB, total - off)
    out.append((off, step))
    off += step
  return out


CH = 8
NCHT = 320
N0 = 19
N1 = NCHT // NS - N0


def _build_agg(with_deg):
  out_type = [jax.ShapeDtypeStruct((NC, NPAD, D), jnp.bfloat16)]
  scratch = [
      pltpu.VMEM_SHARED((NPAD, D), jnp.bfloat16),
      pltpu.SemaphoreType.DMA,
      pltpu.SemaphoreType.DMA,
      pltpu.SemaphoreType.DMA,
      pltpu.SemaphoreType.DMA,
      pltpu.SemaphoreType.DMA,
      pltpu.SemaphoreType.DMA,
  ]
  if with_deg:
    out_type.append(jax.ShapeDtypeStruct((NC, NPAD, DW), jnp.float32))
    scratch += [pltpu.VMEM_SHARED((NPAD, DW), jnp.float32)]

  def body(x_hbm, src_hbm, dst_hbm, *refs):
    if with_deg:
      (ones_hbm, zeros_hbm, agg_out, deg_out, agg_sh, sem0, sem1, sem2,
       sem3, semi, ssem, deg_sh) = refs
    else:
      (agg_out, agg_sh, sem0, sem1, sem2, sem3, semi, ssem) = refs

    tile_scratch = [
        pltpu.VMEM((2, CH, EPB), jnp.int32),
        pltpu.VMEM((2, CH, EPB), jnp.int32),
        pltpu.VMEM((EPB, D), jnp.bfloat16),
        pltpu.VMEM((EPB, D), jnp.bfloat16),
        pltpu.VMEM((EPB, D), jnp.bfloat16),
        pltpu.VMEM((EPB, D), jnp.bfloat16),
    ]
    if with_deg:
      tile_scratch += [
          pltpu.VMEM((EPB, DW), jnp.float32),
          pltpu.VMEM((EPB, DW), jnp.float32),
      ]

    def inner(srcc, dstc, rows0, rows1, rows2, rows3, *deg_refs):
      ones_v, zdeg_v = deg_refs if with_deg else (None, None)
      c = lax.axis_index("c")
      s = lax.axis_index("s")
      nch = jnp.where(c == 0, N0, N1)
      ch0 = jnp.where(c == 0, s * N0, NS * N0 + s * N1)
      zero32 = jnp.zeros((32,), jnp.bfloat16)

      def zrow(i, carry):
        for j in range(D // 32):
          rows0[i, pl.ds(j * 32, 32)] = zero32
        return carry
      lax.fori_loop(0, EPB, zrow, 0)
      base = pl.multiple_of(s * RT, 8)
      for off, step in _zero_chunks(RT):
        pltpu.sync_copy(rows0.at[pl.ds(0, step)],
                        agg_sh.at[pl.ds(base + off, step)])
      if with_deg:
        pltpu.sync_copy(ones_hbm, ones_v)
        pltpu.sync_copy(zeros_hbm, zdeg_v)
        for off, step in _zero_chunks(RT):
          pltpu.sync_copy(zdeg_v.at[pl.ds(0, step)],
                          deg_sh.at[pl.ds(base + off, step)])
      plsc.subcore_barrier()

      bufs = [(rows0, sem0), (rows1, sem1), (rows2, sem2), (rows3, sem3)]
      pltpu.sync_copy(src_hbm.at[ch0], srcc.at[0])
      pltpu.sync_copy(dst_hbm.at[ch0], dstc.at[0])
      pltpu.async_copy(x_hbm.at[srcc.at[0, 0]], rows0, sem0)
      pltpu.async_copy(x_hbm.at[srcc.at[0, 1]], rows1, sem1)

      def chunk_fn(g, carry):
        p = lax.rem(g, 2)
        q = lax.rem(g + 1, 2)

        @pl.when(g + 1 < nch)
        def _():
          pltpu.async_copy(src_hbm.at[ch0 + g + 1], srcc.at[q], semi)
          pltpu.async_copy(dst_hbm.at[ch0 + g + 1], dstc.at[q], semi)

        sd = [None] * CH
        dd = [None] * CH
        for b in range(CH):
          rbuf, sem = bufs[b % 4]
          nbuf, nsem = bufs[(b + 2) % 4]
          if b >= 2:
            sd[b - 2].wait()
            if with_deg:
              dd[b - 2].wait()
          if b + 2 < CH:
            pltpu.async_copy(x_hbm.at[srcc.at[p, b + 2]], nbuf, nsem)
          else:
            @pl.when(g + 1 < nch)
            def _():
              if b + 2 == CH:
                pltpu.make_async_copy(src_hbm.at[ch0], srcc.at[q],
                                      semi).wait()
                pltpu.make_async_copy(dst_hbm.at[ch0], dstc.at[q],
                                      semi).wait()
              pltpu.async_copy(x_hbm.at[srcc.at[q, b + 2 - CH]], nbuf, nsem)
          pltpu.make_async_copy(x_hbm.at[srcc.at[p, b]], rbuf, sem).wait()
          sd[b] = pltpu.async_copy(rbuf, agg_sh.at[dstc.at[p, b]], ssem,
                                   add=True)
          if with_deg:
            dd[b] = pltpu.async_copy(ones_v, deg_sh.at[dstc.at[p, b]], ssem,
                                     add=True)
        sd[CH - 2].wait()
        sd[CH - 1].wait()
        if with_deg:
          dd[CH - 2].wait()
          dd[CH - 1].wait()
        return carry
      lax.fori_loop(0, nch, chunk_fn, 0)

      @pl.when(nch == 0)
      def _():
        pltpu.make_async_copy(x_hbm.at[srcc.at[0, 0]], rows0, sem0).wait()
        pltpu.make_async_copy(x_hbm.at[srcc.at[0, 1]], rows1, sem1).wait()

      plsc.subcore_barrier()
      pltpu.sync_copy(agg_sh.at[pl.ds(base, RT)],
                      agg_out.at[c, pl.ds(base, RT)])
      if with_deg:
        pltpu.sync_copy(deg_sh.at[pl.ds(base, RT)],
                        deg_out.at[c, pl.ds(base, RT)])

    pl.run_scoped(inner, *tile_scratch)

  return pl.kernel(
      body,
      out_type=tuple(out_type) if with_deg else out_type[0],
      mesh=plsc.VectorSubcoreMesh(core_axis_name="c", subcore_axis_name="s"),
      scratch_types=scratch,
      compiler_params=pltpu.CompilerParams(use_tc_tiling_on_sc=False),
  )


def _build_dense(relu, normalize, bf16_copy):
  RB = 1024

  def body(agg_ref, deg_ref, x_ref, wl_ref, wr_ref, b_ref, *rest):
    if bf16_copy:
      rest = rest[1:]
    outs = rest
    deg = deg_ref[0][:, :1] + deg_ref[1][:, :1]
    agg = (agg_ref[0].astype(jnp.float32) + agg_ref[1].astype(jnp.float32))
    mean = agg / jnp.maximum(deg, 1.0)
    y = (jnp.dot(mean, wl_ref[...], preferred_element_type=jnp.float32)
         + jnp.dot(x_ref[...], wr_ref[...], preferred_element_type=jnp.float32)
         + b_ref[...])
    if relu:
      y = jnp.maximum(y, 0.0)
    if normalize:
      nrm = jnp.sqrt(jnp.sum(y * y, axis=1, keepdims=True))
      y = y / jnp.maximum(nrm, 1e-12)
    outs[0][...] = y
    if bf16_copy:
      outs[1][...] = y.astype(jnp.bfloat16)

  out_shape = [jax.ShapeDtypeStruct((NPAD, D), jnp.float32)]
  out_specs = [pl.BlockSpec((RB, D), lambda i: (i, 0))]
  in_specs = [
      pl.BlockSpec((NC, RB, D), lambda i: (0, i, 0)),
      pl.BlockSpec((NC, RB, DW), lambda i: (0, i, 0)),
      pl.BlockSpec((RB, D), lambda i: (i, 0)),
      pl.BlockSpec((D, D), lambda i: (0, 0)),
      pl.BlockSpec((D, D), lambda i: (0, 0)),
      pl.BlockSpec((1, D), lambda i: (0, 0)),
  ]
  aliases = {}
  if bf16_copy:
    out_shape.append(jax.ShapeDtypeStruct((NPAD, D), jnp.bfloat16))
    out_specs.append(pl.BlockSpec((RB, D), lambda i: (i, 0)))
    in_specs.append(pl.BlockSpec((RB, D), lambda i: (i, 0)))
    aliases = {6: 1}
  return pl.pallas_call(
      body,
      grid=(pl.cdiv(NPAD, RB),),
      in_specs=in_specs,
      out_specs=tuple(out_specs),
      out_shape=tuple(out_shape),
      input_output_aliases=aliases,
  )


_agg_deg = _build_agg(True)
_agg = _build_agg(False)
_dense_relu = _build_dense(True, False, True)
_dense_norm = _build_dense(False, True, False)


@jax.jit
def kernel(x, edge_index, Wl1, bl1, Wr1, Wl2, bl2, Wr2):
  e = edge_index.shape[1]
  epad = (NCHT + 1) * CH * EPB
  src = jnp.concatenate(
      [edge_index[0], jnp.zeros((epad - e,), jnp.int32)]).reshape(
          NCHT + 1, CH, EPB)
  dst = jnp.concatenate(
      [edge_index[1], jnp.full((epad - e,), N, jnp.int32)]).reshape(
          NCHT + 1, CH, EPB)
  x_p = jnp.pad(x, ((0, NPAD - N), (0, 0)))

  ones8 = jnp.ones((EPB, DW), jnp.float32)
  zeros8 = jnp.zeros((EPB, DW), jnp.float32)
  x_b = x_p.astype(jnp.bfloat16)
  agg1, deg = _agg_deg(x_b, src, dst, ones8, zeros8)
  h, h_b = _dense_relu(agg1, deg, x_p, Wl1.T, Wr1.T, bl1.reshape(1, D), x_b)
  agg2 = _agg(h_b, src, dst)
  y, = _dense_norm(agg2, deg, h, Wl2.T, Wr2.T, bl2.reshape(1, D))
  return y[:N]
